# baseline (speedup 1.0000x reference)
import math

import jax
import jax.numpy as jnp
from jax.experimental import pallas as pl
from jax.experimental.pallas import tpu as pltpu

_B, _R, _N, _L, _D, _H, _HID, _K = 2, 2, 128, 50, 512, 16, 2048, 512
_HD = _D // _H
_TI = 16
_D2 = _D // 2


def _ln(x, g, b):
    m = x.mean(-1, keepdims=True)
    v = x.var(-1, keepdims=True)
    return (x - m) / jnp.sqrt(v + 1e-5) * g + b


def _cross_block(x, y, p):
    Bn, Nn, Dd = x.shape
    Ll = y.shape[1]
    hd = Dd // _H
    q = (x @ p['Wq'] + p['bq']).reshape(Bn, Nn, _H, hd).transpose(0, 2, 1, 3)
    k = (y @ p['Wk'] + p['bk']).reshape(Bn, Ll, _H, hd).transpose(0, 2, 1, 3)
    v = (y @ p['Wv'] + p['bv']).reshape(Bn, Ll, _H, hd).transpose(0, 2, 1, 3)
    att = jax.nn.softmax(jnp.einsum('bhnd,bhld->bhnl', q, k) / math.sqrt(hd),
                         axis=-1)
    o = jnp.einsum('bhnl,bhld->bhnd', att, v).transpose(0, 2, 1, 3) \
        .reshape(Bn, Nn, Dd)
    h = _ln(x + o @ p['Wo'] + p['bo'], p['ln1_g'], p['ln1_b'])
    f = jax.nn.relu(h @ p['W1'] + p['b1']) @ p['W2'] + p['b2']
    return _ln(h + f, p['ln2_g'], p['ln2_b'])


def _logits_body(obf_ref, geo_ref, cw1, cb1, cw2, cb2, out_ref):
    r = pl.program_id(2)
    ib = pl.program_id(1)
    ob = obf_ref[0, 0]
    cols = []
    for i in range(_TI):
        brow = obf_ref[0, 0, pl.ds(ib * _TI + i, 1), :]
        feats = ob * brow + geo_ref[0, 0, i]
        z = jax.nn.relu(jnp.dot(feats, cw1[...]) + cb1[...])
        col = jnp.dot(z, cw2[...]) + cb2[0, 0]
        cols.append(col)
    blk = jnp.concatenate(cols, axis=1) * (1.0 / _R)

    @pl.when(r == 0)
    def _():
        out_ref[0, 0] = blk

    @pl.when(r != 0)
    def _():
        out_ref[0, 0] += blk


def _gather_body(ia_ref, jb_ref, oa_ref, ob_ref, geo_ref, out_ref):
    acc = jnp.zeros((1, _D), jnp.float32)
    for r in range(_R):
        acc = acc + oa_ref[0, r, 0] * ob_ref[0, r, 0] + geo_ref[0, r, 0]
    out_ref[0, 0] = acc * (1.0 / _R)


def kernel(obj_feats, geo_feats, text_feats, box_feats, params, real_nums):
    del real_nums
    f32 = jnp.float32
    row = lambda a: a.reshape(1, -1)

    of = obj_feats[:, None] + box_feats
    obf = []
    for bi in range(_B):
        vt = jnp.broadcast_to(text_feats[bi][None], (_R, _L, _D))
        obf.append(_cross_block(of[bi], vt, params))
    obf = jnp.stack(obf)

    wspec3 = lambda shp: pl.BlockSpec(shp, lambda b, ib, r: (0,) * len(shp))
    logits_t = pl.pallas_call(
        _logits_body,
        grid=(_B, _N // _TI, _R),
        in_specs=[
            pl.BlockSpec((1, 1, _N, _D), lambda b, ib, r: (b, r, 0, 0)),
            pl.BlockSpec((1, 1, _TI, _N, _D), lambda b, ib, r: (b, r, ib, 0, 0)),
            wspec3((_D, _D2)), wspec3((1, _D2)),
            wspec3((_D2, 1)), wspec3((1, 1)),
        ],
        out_specs=pl.BlockSpec((1, 1, _N, _TI), lambda b, ib, r: (b, ib, 0, 0)),
        out_shape=jax.ShapeDtypeStruct((_B, _N // _TI, _N, _TI), f32),
        compiler_params=pltpu.CompilerParams(
            dimension_semantics=("parallel", "parallel", "arbitrary")),
    )(obf, geo_feats,
      params['cW1'], row(params['cb1']),
      params['cW2'], params['cb2'].reshape(1, 1))
    logits = jnp.transpose(logits_t, (0, 1, 3, 2)).reshape(_B, _N, _N)

    mask = jnp.ones((_N, _N), f32) - jnp.eye(_N, dtype=f32)
    topk = []
    for bi in range(_B):
        pair_probs = jax.nn.sigmoid(logits[bi].reshape(_N * _N)) * mask.ravel()
        _, tk = jax.lax.top_k(pair_probs, _K)
        topk.append(tk)
    topk = jnp.stack(topk)
    ia = (topk // _N).astype(jnp.int32)
    jb = (topk % _N).astype(jnp.int32)

    obf_rows = obf.reshape(_B, _R, _N, 1, _D)
    geo_rows = geo_feats.reshape(_B, _R, _N * _N, 1, _D)
    out_feats = pl.pallas_call(
        _gather_body,
        grid_spec=pltpu.PrefetchScalarGridSpec(
            num_scalar_prefetch=2,
            grid=(_B, _K),
            in_specs=[
                pl.BlockSpec((1, _R, 1, 1, _D),
                             lambda b, t, ia_r, jb_r: (b, 0, ia_r[b, t], 0, 0)),
                pl.BlockSpec((1, _R, 1, 1, _D),
                             lambda b, t, ia_r, jb_r: (b, 0, jb_r[b, t], 0, 0)),
                pl.BlockSpec((1, _R, 1, 1, _D),
                             lambda b, t, ia_r, jb_r:
                             (b, 0, ia_r[b, t] * _N + jb_r[b, t], 0, 0)),
            ],
            out_specs=pl.BlockSpec((1, 1, 1, _D),
                                   lambda b, t, ia_r, jb_r: (b, t, 0, 0)),
        ),
        out_shape=jax.ShapeDtypeStruct((_B, _K, 1, _D), f32),
    )(ia, jb, obf_rows, obf_rows, geo_rows)
    out_feats = out_feats.reshape(_B, _K, _D)

    out_inds = jnp.stack([jb, ia], axis=-1)
    return logits, out_feats, out_inds


# speedup vs baseline: 1.6607x; 1.6607x over previous
import math

import jax
import jax.numpy as jnp
from jax.experimental import pallas as pl
from jax.experimental.pallas import tpu as pltpu

_B, _R, _N, _L, _D, _H, _HID, _K = 2, 2, 128, 50, 512, 16, 2048, 512
_HD = _D // _H
_TI = 16
_D2 = _D // 2


def _ln(x, g, b):
    m = x.mean(-1, keepdims=True)
    v = x.var(-1, keepdims=True)
    return (x - m) / jnp.sqrt(v + 1e-5) * g + b


def _cross_block(x, y, p):
    Bn, Nn, Dd = x.shape
    Ll = y.shape[1]
    hd = Dd // _H
    q = (x @ p['Wq'] + p['bq']).reshape(Bn, Nn, _H, hd).transpose(0, 2, 1, 3)
    k = (y @ p['Wk'] + p['bk']).reshape(Bn, Ll, _H, hd).transpose(0, 2, 1, 3)
    v = (y @ p['Wv'] + p['bv']).reshape(Bn, Ll, _H, hd).transpose(0, 2, 1, 3)
    att = jax.nn.softmax(jnp.einsum('bhnd,bhld->bhnl', q, k) / math.sqrt(hd),
                         axis=-1)
    o = jnp.einsum('bhnl,bhld->bhnd', att, v).transpose(0, 2, 1, 3) \
        .reshape(Bn, Nn, Dd)
    h = _ln(x + o @ p['Wo'] + p['bo'], p['ln1_g'], p['ln1_b'])
    f = jax.nn.relu(h @ p['W1'] + p['b1']) @ p['W2'] + p['b2']
    return _ln(h + f, p['ln2_g'], p['ln2_b'])


def _logits_body(obf_ref, geo_ref, cw1, cb1, cw2, cb2, out_ref):
    r = pl.program_id(2)
    ib = pl.program_id(1)
    ob = obf_ref[0, 0]
    cols = []
    for i in range(_TI):
        brow = obf_ref[0, 0, pl.ds(ib * _TI + i, 1), :]
        feats = ob * brow + geo_ref[0, 0, i]
        z = jax.nn.relu(jnp.dot(feats, cw1[...]) + cb1[...])
        col = jnp.dot(z, cw2[...]) + cb2[0, 0]
        cols.append(col)
    blk = jnp.concatenate(cols, axis=1) * (1.0 / _R)

    @pl.when(r == 0)
    def _():
        out_ref[0, 0] = blk

    @pl.when(r != 0)
    def _():
        out_ref[0, 0] += blk


_G = 16


def _gather_body(ia_ref, jb_ref, obf_ref, *geo_and_out):
    geo_refs, out_ref = geo_and_out[:-1], geo_and_out[-1]
    b = pl.program_id(0)
    t = pl.program_id(1)
    for g in range(_G):
        idx = t * _G + g
        acc = jnp.zeros((1, _D), jnp.float32)
        for r in range(_R):
            arow = obf_ref[0, r, pl.ds(ia_ref[b, idx], 1), :]
            brow = obf_ref[0, r, pl.ds(jb_ref[b, idx], 1), :]
            acc = acc + arow * brow + geo_refs[g][0, r, 0]
        out_ref[0, 0, pl.ds(g, 1), :] = acc * (1.0 / _R)


def kernel(obj_feats, geo_feats, text_feats, box_feats, params, real_nums):
    del real_nums
    f32 = jnp.float32
    row = lambda a: a.reshape(1, -1)

    of = obj_feats[:, None] + box_feats
    obf = []
    for bi in range(_B):
        vt = jnp.broadcast_to(text_feats[bi][None], (_R, _L, _D))
        obf.append(_cross_block(of[bi], vt, params))
    obf = jnp.stack(obf)

    wspec3 = lambda shp: pl.BlockSpec(shp, lambda b, ib, r: (0,) * len(shp))
    logits_t = pl.pallas_call(
        _logits_body,
        grid=(_B, _N // _TI, _R),
        in_specs=[
            pl.BlockSpec((1, 1, _N, _D), lambda b, ib, r: (b, r, 0, 0)),
            pl.BlockSpec((1, 1, _TI, _N, _D), lambda b, ib, r: (b, r, ib, 0, 0)),
            wspec3((_D, _D2)), wspec3((1, _D2)),
            wspec3((_D2, 1)), wspec3((1, 1)),
        ],
        out_specs=pl.BlockSpec((1, 1, _N, _TI), lambda b, ib, r: (b, ib, 0, 0)),
        out_shape=jax.ShapeDtypeStruct((_B, _N // _TI, _N, _TI), f32),
        compiler_params=pltpu.CompilerParams(
            dimension_semantics=("parallel", "parallel", "arbitrary")),
    )(obf, geo_feats,
      params['cW1'], row(params['cb1']),
      params['cW2'], params['cb2'].reshape(1, 1))
    logits = jnp.transpose(logits_t, (0, 1, 3, 2)).reshape(_B, _N, _N)

    mask = jnp.ones((_N, _N), f32) - jnp.eye(_N, dtype=f32)
    topk = []
    for bi in range(_B):
        pair_probs = jax.nn.sigmoid(logits[bi].reshape(_N * _N)) * mask.ravel()
        _, tk = jax.lax.top_k(pair_probs, _K)
        topk.append(tk)
    topk = jnp.stack(topk)
    ia = (topk // _N).astype(jnp.int32)
    jb = (topk % _N).astype(jnp.int32)

    geo_rows = geo_feats.reshape(_B, _R, _N * _N, 1, _D)
    geo_specs = [
        pl.BlockSpec((1, _R, 1, 1, _D),
                     lambda b, t, ia_r, jb_r, g=g:
                     (b, 0, ia_r[b, t * _G + g] * _N + jb_r[b, t * _G + g],
                      0, 0))
        for g in range(_G)
    ]
    out_feats = pl.pallas_call(
        _gather_body,
        grid_spec=pltpu.PrefetchScalarGridSpec(
            num_scalar_prefetch=2,
            grid=(_B, _K // _G),
            in_specs=[
                pl.BlockSpec((1, _R, _N, _D),
                             lambda b, t, ia_r, jb_r: (b, 0, 0, 0)),
            ] + geo_specs,
            out_specs=pl.BlockSpec((1, 1, _G, _D),
                                   lambda b, t, ia_r, jb_r: (b, t, 0, 0)),
        ),
        out_shape=jax.ShapeDtypeStruct((_B, _K // _G, _G, _D), f32),
    )(ia, jb, obf, *([geo_rows] * _G))
    out_feats = out_feats.reshape(_B, _K, _D)

    out_inds = jnp.stack([jb, ia], axis=-1)
    return logits, out_feats, out_inds


# speedup vs baseline: 1.6618x; 1.0007x over previous
import math

import jax
import jax.numpy as jnp
from jax.experimental import pallas as pl
from jax.experimental.pallas import tpu as pltpu

_B, _R, _N, _L, _D, _H, _HID, _K = 2, 2, 128, 50, 512, 16, 2048, 512
_HD = _D // _H
_TI = 16
_D2 = _D // 2


def _ln(x, g, b):
    m = x.mean(-1, keepdims=True)
    v = x.var(-1, keepdims=True)
    return (x - m) / jnp.sqrt(v + 1e-5) * g + b


def _cross_block(x, y, p):
    Bn, Nn, Dd = x.shape
    Ll = y.shape[1]
    hd = Dd // _H
    q = (x @ p['Wq'] + p['bq']).reshape(Bn, Nn, _H, hd).transpose(0, 2, 1, 3)
    k = (y @ p['Wk'] + p['bk']).reshape(Bn, Ll, _H, hd).transpose(0, 2, 1, 3)
    v = (y @ p['Wv'] + p['bv']).reshape(Bn, Ll, _H, hd).transpose(0, 2, 1, 3)
    att = jax.nn.softmax(jnp.einsum('bhnd,bhld->bhnl', q, k) / math.sqrt(hd),
                         axis=-1)
    o = jnp.einsum('bhnl,bhld->bhnd', att, v).transpose(0, 2, 1, 3) \
        .reshape(Bn, Nn, Dd)
    h = _ln(x + o @ p['Wo'] + p['bo'], p['ln1_g'], p['ln1_b'])
    f = jax.nn.relu(h @ p['W1'] + p['b1']) @ p['W2'] + p['b2']
    return _ln(h + f, p['ln2_g'], p['ln2_b'])


def _logits_body(obf_ref, geo_ref, cw1, cb1, cw2, cb2, out_ref):
    r = pl.program_id(2)
    ib = pl.program_id(1)
    ob = obf_ref[0, 0]
    cols = []
    for i in range(_TI):
        brow = obf_ref[0, 0, pl.ds(ib * _TI + i, 1), :]
        feats = ob * brow + geo_ref[0, 0, i]
        z = jax.nn.relu(jnp.dot(feats, cw1[...]) + cb1[...])
        col = jnp.dot(z, cw2[...]) + cb2[0, 0]
        cols.append(col)
    blk = jnp.concatenate(cols, axis=1) * (1.0 / _R)

    @pl.when(r == 0)
    def _():
        out_ref[0, 0] = blk

    @pl.when(r != 0)
    def _():
        out_ref[0, 0] += blk


_G = 32


def _gather_body(ia_ref, jb_ref, obf_ref, *geo_and_out):
    geo_refs, out_ref = geo_and_out[:-1], geo_and_out[-1]
    b = pl.program_id(0)
    t = pl.program_id(1)
    for g in range(_G):
        idx = t * _G + g
        acc = jnp.zeros((1, _D), jnp.float32)
        for r in range(_R):
            arow = obf_ref[0, r, pl.ds(ia_ref[b, idx], 1), :]
            brow = obf_ref[0, r, pl.ds(jb_ref[b, idx], 1), :]
            acc = acc + arow * brow + geo_refs[g][0, r, 0]
        out_ref[0, 0, pl.ds(g, 1), :] = acc * (1.0 / _R)


def kernel(obj_feats, geo_feats, text_feats, box_feats, params, real_nums):
    del real_nums
    f32 = jnp.float32
    row = lambda a: a.reshape(1, -1)

    of = obj_feats[:, None] + box_feats
    obf = []
    for bi in range(_B):
        vt = jnp.broadcast_to(text_feats[bi][None], (_R, _L, _D))
        obf.append(_cross_block(of[bi], vt, params))
    obf = jnp.stack(obf)

    wspec3 = lambda shp: pl.BlockSpec(shp, lambda b, ib, r: (0,) * len(shp))
    logits_t = pl.pallas_call(
        _logits_body,
        grid=(_B, _N // _TI, _R),
        in_specs=[
            pl.BlockSpec((1, 1, _N, _D), lambda b, ib, r: (b, r, 0, 0)),
            pl.BlockSpec((1, 1, _TI, _N, _D), lambda b, ib, r: (b, r, ib, 0, 0)),
            wspec3((_D, _D2)), wspec3((1, _D2)),
            wspec3((_D2, 1)), wspec3((1, 1)),
        ],
        out_specs=pl.BlockSpec((1, 1, _N, _TI), lambda b, ib, r: (b, ib, 0, 0)),
        out_shape=jax.ShapeDtypeStruct((_B, _N // _TI, _N, _TI), f32),
        compiler_params=pltpu.CompilerParams(
            dimension_semantics=("parallel", "parallel", "arbitrary")),
    )(obf, geo_feats,
      params['cW1'], row(params['cb1']),
      params['cW2'], params['cb2'].reshape(1, 1))
    logits = jnp.transpose(logits_t, (0, 1, 3, 2)).reshape(_B, _N, _N)

    mask = jnp.ones((_N, _N), f32) - jnp.eye(_N, dtype=f32)
    topk = []
    for bi in range(_B):
        pair_probs = jax.nn.sigmoid(logits[bi].reshape(_N * _N)) * mask.ravel()
        _, tk = jax.lax.top_k(pair_probs, _K)
        topk.append(tk)
    topk = jnp.stack(topk)
    ia = (topk // _N).astype(jnp.int32)
    jb = (topk % _N).astype(jnp.int32)

    geo_rows = geo_feats.reshape(_B, _R, _N * _N, 1, _D)
    geo_specs = [
        pl.BlockSpec((1, _R, 1, 1, _D),
                     lambda b, t, ia_r, jb_r, g=g:
                     (b, 0, ia_r[b, t * _G + g] * _N + jb_r[b, t * _G + g],
                      0, 0))
        for g in range(_G)
    ]
    out_feats = pl.pallas_call(
        _gather_body,
        grid_spec=pltpu.PrefetchScalarGridSpec(
            num_scalar_prefetch=2,
            grid=(_B, _K // _G),
            in_specs=[
                pl.BlockSpec((1, _R, _N, _D),
                             lambda b, t, ia_r, jb_r: (b, 0, 0, 0)),
            ] + geo_specs,
            out_specs=pl.BlockSpec((1, 1, _G, _D),
                                   lambda b, t, ia_r, jb_r: (b, t, 0, 0)),
        ),
        out_shape=jax.ShapeDtypeStruct((_B, _K // _G, _G, _D), f32),
    )(ia, jb, obf, *([geo_rows] * _G))
    out_feats = out_feats.reshape(_B, _K, _D)

    out_inds = jnp.stack([jb, ia], axis=-1)
    return logits, out_feats, out_inds


# speedup vs baseline: 3.6625x; 2.2039x over previous
import math

import jax
import jax.numpy as jnp
from jax.experimental import pallas as pl
from jax.experimental.pallas import tpu as pltpu

_B, _R, _N, _L, _D, _H, _HID, _K = 2, 2, 128, 50, 512, 16, 2048, 512
_HD = _D // _H
_TI = 16
_D2 = _D // 2


def _ln(x, g, b):
    m = x.mean(-1, keepdims=True)
    v = x.var(-1, keepdims=True)
    return (x - m) / jnp.sqrt(v + 1e-5) * g + b


def _cross_block(x, y, p):
    Bn, Nn, Dd = x.shape
    Ll = y.shape[1]
    hd = Dd // _H
    q = (x @ p['Wq'] + p['bq']).reshape(Bn, Nn, _H, hd).transpose(0, 2, 1, 3)
    k = (y @ p['Wk'] + p['bk']).reshape(Bn, Ll, _H, hd).transpose(0, 2, 1, 3)
    v = (y @ p['Wv'] + p['bv']).reshape(Bn, Ll, _H, hd).transpose(0, 2, 1, 3)
    att = jax.nn.softmax(jnp.einsum('bhnd,bhld->bhnl', q, k) / math.sqrt(hd),
                         axis=-1)
    o = jnp.einsum('bhnl,bhld->bhnd', att, v).transpose(0, 2, 1, 3) \
        .reshape(Bn, Nn, Dd)
    h = _ln(x + o @ p['Wo'] + p['bo'], p['ln1_g'], p['ln1_b'])
    f = jax.nn.relu(h @ p['W1'] + p['b1']) @ p['W2'] + p['b2']
    return _ln(h + f, p['ln2_g'], p['ln2_b'])


def _logits_body(obf_ref, geo_ref, cw1, cb1, cw2, cb2, out_ref):
    r = pl.program_id(2)
    ib = pl.program_id(1)
    ob = obf_ref[0, 0]
    cols = []
    for i in range(_TI):
        brow = obf_ref[0, 0, pl.ds(ib * _TI + i, 1), :]
        feats = ob * brow + geo_ref[0, 0, i]
        z = jax.nn.relu(jnp.dot(feats, cw1[...]) + cb1[...])
        col = jnp.dot(z, cw2[...]) + cb2[0, 0]
        cols.append(col)
    blk = jnp.concatenate(cols, axis=1) * (1.0 / _R)

    @pl.when(r == 0)
    def _():
        out_ref[0, 0] = blk

    @pl.when(r != 0)
    def _():
        out_ref[0, 0] += blk


_G = 16


def _gather_body(ia_ref, jb_ref, obf_ref, gsel_ref, out_ref):
    b = pl.program_id(0)
    t = pl.program_id(1)
    for g in range(_G):
        idx = t * _G + g
        acc = jnp.zeros((1, _D), jnp.float32)
        for r in range(_R):
            arow = obf_ref[0, r, pl.ds(ia_ref[b, idx], 1), :]
            brow = obf_ref[0, r, pl.ds(jb_ref[b, idx], 1), :]
            acc = acc + arow * brow + gsel_ref[0, r, pl.ds(g, 1), :]
        out_ref[0, 0, pl.ds(g, 1), :] = acc * (1.0 / _R)


def kernel(obj_feats, geo_feats, text_feats, box_feats, params, real_nums):
    del real_nums
    f32 = jnp.float32
    row = lambda a: a.reshape(1, -1)

    of = obj_feats[:, None] + box_feats
    obf = []
    for bi in range(_B):
        vt = jnp.broadcast_to(text_feats[bi][None], (_R, _L, _D))
        obf.append(_cross_block(of[bi], vt, params))
    obf = jnp.stack(obf)

    wspec3 = lambda shp: pl.BlockSpec(shp, lambda b, ib, r: (0,) * len(shp))
    logits_t = pl.pallas_call(
        _logits_body,
        grid=(_B, _N // _TI, _R),
        in_specs=[
            pl.BlockSpec((1, 1, _N, _D), lambda b, ib, r: (b, r, 0, 0)),
            pl.BlockSpec((1, 1, _TI, _N, _D), lambda b, ib, r: (b, r, ib, 0, 0)),
            wspec3((_D, _D2)), wspec3((1, _D2)),
            wspec3((_D2, 1)), wspec3((1, 1)),
        ],
        out_specs=pl.BlockSpec((1, 1, _N, _TI), lambda b, ib, r: (b, ib, 0, 0)),
        out_shape=jax.ShapeDtypeStruct((_B, _N // _TI, _N, _TI), f32),
        compiler_params=pltpu.CompilerParams(
            dimension_semantics=("parallel", "parallel", "arbitrary")),
    )(obf, geo_feats,
      params['cW1'], row(params['cb1']),
      params['cW2'], params['cb2'].reshape(1, 1))
    logits = jnp.transpose(logits_t, (0, 1, 3, 2)).reshape(_B, _N, _N)

    mask = jnp.ones((_N, _N), f32) - jnp.eye(_N, dtype=f32)
    topk = []
    for bi in range(_B):
        pair_probs = jax.nn.sigmoid(logits[bi].reshape(_N * _N)) * mask.ravel()
        _, tk = jax.lax.top_k(pair_probs, _K)
        topk.append(tk)
    topk = jnp.stack(topk)
    ia = (topk // _N).astype(jnp.int32)
    jb = (topk % _N).astype(jnp.int32)

    gsel = jnp.take_along_axis(geo_feats.reshape(_B, _R, _N * _N, _D),
                               topk[:, None, :, None], axis=2)
    out_feats = pl.pallas_call(
        _gather_body,
        grid_spec=pltpu.PrefetchScalarGridSpec(
            num_scalar_prefetch=2,
            grid=(_B, _K // _G),
            in_specs=[
                pl.BlockSpec((1, _R, _N, _D),
                             lambda b, t, ia_r, jb_r: (b, 0, 0, 0)),
                pl.BlockSpec((1, _R, _G, _D),
                             lambda b, t, ia_r, jb_r: (b, 0, t, 0)),
            ],
            out_specs=pl.BlockSpec((1, 1, _G, _D),
                                   lambda b, t, ia_r, jb_r: (b, t, 0, 0)),
        ),
        out_shape=jax.ShapeDtypeStruct((_B, _K // _G, _G, _D), f32),
    )(ia, jb, obf, gsel)
    out_feats = out_feats.reshape(_B, _K, _D)

    out_inds = jnp.stack([jb, ia], axis=-1)
    return logits, out_feats, out_inds
